# baseline (speedup 1.0000x reference)
import functools

import jax
import jax.numpy as jnp
from jax import lax
from jax.experimental import pallas as pl
from jax.experimental.pallas import tpu as pltpu
from jax.experimental.pallas import tpu_sc as plsc

_N = 10000
_E = 320000
_D = 128
_H = 128
_C = 16
_G = 64

_NC = 2
_NS = 16
_CHUNK = 128
_CPT = 80
_E_PAD = _NC * _NS * _CPT * _CHUNK
_N_PAD = 10240
_ROWS_PER_TILE = _N_PAD // _NS


def _seg_sum_body(feat, y_hbm, rh_hbm, src_hbm, dst_hbm, out_hbm,
                  acc_sh, src0, dst0, src1, dst1, rows0, rows1,
                  sem_i0, sem_i1, sem_g0, sem_g1):
    c = lax.axis_index("c")
    s = lax.axis_index("s")
    w = c * _NS + s
    srcs = (src0, src1)
    dsts = (dst0, dst1)
    rows = (rows0, rows1)
    isems = (sem_i0, sem_i1)
    gsems = (sem_g0, sem_g1)
    ebase = w * (_CPT * _CHUNK)

    base_r = s * _ROWS_PER_TILE
    last_full = (_N - (_NS - 1) * _ROWS_PER_TILE)

    @pl.when(s < _NS - 1)
    def _():
        pltpu.sync_copy(rh_hbm.at[pl.ds(base_r, _ROWS_PER_TILE)],
                        acc_sh.at[pl.ds(base_r, _ROWS_PER_TILE)])

    @pl.when(s == _NS - 1)
    def _():
        lr = (_NS - 1) * _ROWS_PER_TILE
        pltpu.sync_copy(rh_hbm.at[pl.ds(lr, last_full)],
                        acc_sh.at[pl.ds(lr, last_full)])

    plsc.subcore_barrier()

    def load_idx(j, b):
        off = pl.multiple_of(ebase + j * _CHUNK, 8)
        h0 = pltpu.async_copy(src_hbm.at[pl.ds(off, _CHUNK)], srcs[b], isems[b])
        h1 = pltpu.async_copy(dst_hbm.at[pl.ds(off, _CHUNK)], dsts[b], isems[b])
        return h0, h1

    def step(j, b, has_next):
        nb = 1 - b
        if has_next:
            i0, i1 = load_idx(j + 1, nb)
        pltpu.make_async_copy(y_hbm.at[srcs[b]], rows[b], gsems[b]).wait()
        if has_next:
            i0.wait()
            i1.wait()
            pltpu.async_copy(y_hbm.at[srcs[nb]], rows[nb], gsems[nb])
        pltpu.sync_copy(rows[b], acc_sh.at[dsts[b]], add=True)

    h0, h1 = load_idx(0, 0)
    h0.wait()
    h1.wait()
    pltpu.async_copy(y_hbm.at[src0], rows0, sem_g0)

    def body(k, carry):
        step(2 * k, 0, True)
        step(2 * k + 1, 1, True)
        return carry

    lax.fori_loop(0, (_CPT - 2) // 2, body, 0)
    step(_CPT - 2, 0, True)
    step(_CPT - 1, 1, False)
    plsc.subcore_barrier()

    row_off = c * _N_PAD + base_r
    pltpu.sync_copy(acc_sh.at[pl.ds(base_r, _ROWS_PER_TILE)],
                    out_hbm.at[pl.ds(row_off, _ROWS_PER_TILE)])


@functools.lru_cache(maxsize=None)
def _make_seg_sum(feat):
    mesh = plsc.VectorSubcoreMesh(core_axis_name="c", subcore_axis_name="s",
                                  num_cores=_NC, num_subcores=_NS)
    return pl.kernel(
        functools.partial(_seg_sum_body, feat),
        out_type=jax.ShapeDtypeStruct((_NC * _N_PAD, feat), jnp.float32),
        mesh=mesh,
        scratch_types=[
            pltpu.VMEM_SHARED((_N_PAD, feat), jnp.float32),
            pltpu.VMEM((_CHUNK,), jnp.int32),
            pltpu.VMEM((_CHUNK,), jnp.int32),
            pltpu.VMEM((_CHUNK,), jnp.int32),
            pltpu.VMEM((_CHUNK,), jnp.int32),
            pltpu.VMEM((_CHUNK, feat), jnp.float32),
            pltpu.VMEM((_CHUNK, feat), jnp.float32),
            pltpu.SemaphoreType.DMA,
            pltpu.SemaphoreType.DMA,
            pltpu.SemaphoreType.DMA,
            pltpu.SemaphoreType.DMA,
        ],
    )


_BLK = 1000
_GRID = _N // _BLK


def _front_body(x_ref, wrel_ref, wroot_ref, b_ref, y_ref, r_ref):
    xb = x_ref[...]
    y_ref[...] = jnp.dot(xb, wrel_ref[...], preferred_element_type=jnp.float32)
    r_ref[...] = 0.5 * (jnp.dot(xb, wroot_ref[...],
                                preferred_element_type=jnp.float32)
                        + b_ref[...])


def _mid_body(fo, acc_a_ref, acc_b_ref, wrel_ref, wroot_ref, b_ref,
              y_ref, r2_ref):
    h = jnp.maximum(acc_a_ref[...] + acc_b_ref[...], 0.0)
    y_ref[...] = jnp.dot(h, wrel_ref[...], preferred_element_type=jnp.float32)
    r2_ref[...] = 0.5 * (jnp.dot(h, wroot_ref[...],
                                 preferred_element_type=jnp.float32)
                         + b_ref[...])


def _make_front(fin, fo):
    return pl.pallas_call(
        _front_body,
        grid=(_GRID,),
        in_specs=[
            pl.BlockSpec((_BLK, fin), lambda i: (i, 0)),
            pl.BlockSpec((fin, fo), lambda i: (0, 0)),
            pl.BlockSpec((fin, fo), lambda i: (0, 0)),
            pl.BlockSpec((1, fo), lambda i: (0, 0)),
        ],
        out_specs=[
            pl.BlockSpec((_BLK, fo), lambda i: (i, 0)),
            pl.BlockSpec((_BLK, fo), lambda i: (i, 0)),
        ],
        out_shape=[
            jax.ShapeDtypeStruct((_N, fo), jnp.float32),
            jax.ShapeDtypeStruct((_N, fo), jnp.float32),
        ],
    )


def _make_mid(fin, fo):
    return pl.pallas_call(
        functools.partial(_mid_body, fo),
        grid=(_GRID,),
        in_specs=[
            pl.BlockSpec((_BLK, fin), lambda i: (i, 0)),
            pl.BlockSpec((_BLK, fin), lambda i: (i, 0)),
            pl.BlockSpec((fin, fo), lambda i: (0, 0)),
            pl.BlockSpec((fin, fo), lambda i: (0, 0)),
            pl.BlockSpec((1, fo), lambda i: (0, 0)),
        ],
        out_specs=[
            pl.BlockSpec((_BLK, fo), lambda i: (i, 0)),
            pl.BlockSpec((_BLK, fo), lambda i: (i, 0)),
        ],
        out_shape=[
            jax.ShapeDtypeStruct((_N, fo), jnp.float32),
            jax.ShapeDtypeStruct((_N, fo), jnp.float32),
        ],
    )


_front128 = _make_front(_D, _H)
_mid128 = _make_mid(_H, _H)


def _pool_body(acc_a_ref, acc_b_ref, batch_ref, out_ref, cnt_ref):
    i = pl.program_id(0)
    t = (acc_a_ref[...] + acc_b_ref[...])[:, :_C]
    bvec = batch_ref[0, 0, :]
    onehot = (bvec[:, None] ==
              lax.broadcasted_iota(jnp.int32, (_BLK, _G), 1)).astype(jnp.float32)
    dims = (((0,), (0,)), ((), ()))
    sums = lax.dot_general(onehot, t, dims, preferred_element_type=jnp.float32)
    cnts = lax.dot_general(onehot, jnp.ones((_BLK, _C), jnp.float32), dims,
                           preferred_element_type=jnp.float32)

    @pl.when(i == 0)
    def _():
        out_ref[...] = sums
        cnt_ref[...] = cnts

    @pl.when(i > 0)
    def _():
        out_ref[...] += sums
        cnt_ref[...] += cnts

    @pl.when(i == pl.num_programs(0) - 1)
    def _():
        out_ref[...] = out_ref[...] / jnp.maximum(cnt_ref[...], 1.0)


_pool = pl.pallas_call(
    _pool_body,
    grid=(_GRID,),
    in_specs=[
        pl.BlockSpec((_BLK, _H), lambda i: (i, 0)),
        pl.BlockSpec((_BLK, _H), lambda i: (i, 0)),
        pl.BlockSpec((1, 1, _BLK), lambda i: (i, 0, 0)),
    ],
    out_specs=pl.BlockSpec((_G, _C), lambda i: (0, 0)),
    out_shape=jax.ShapeDtypeStruct((_G, _C), jnp.float32),
    scratch_shapes=[pltpu.VMEM((_G, _C), jnp.float32)],
)


def kernel(x, edge_index, batch, W1_rel, W1_root, b1, W2_rel, W2_root, b2,
           W3_rel, W3_root, b3):
    npad = _E_PAD - _E
    src = jnp.concatenate([edge_index[0],
                           jnp.zeros((npad,), jnp.int32)])
    dst = jnp.concatenate([edge_index[1],
                           jnp.full((npad,), _N_PAD - 1, jnp.int32)])
    seg128 = _make_seg_sum(_H)
    W3_rel_p = jnp.pad(W3_rel, ((0, 0), (0, _H - _C)))
    W3_root_p = jnp.pad(W3_root, ((0, 0), (0, _H - _C)))
    b3_p = jnp.pad(b3, (0, _H - _C))
    y1, rh1 = _front128(x, W1_rel, W1_root, b1.reshape(1, _H))
    acc1 = seg128(y1, rh1, src, dst)
    y2, rh2 = _mid128(acc1[:_N], acc1[_N_PAD:_N_PAD + _N], W2_rel, W2_root,
                      b2.reshape(1, _H))
    acc2 = seg128(y2, rh2, src, dst)
    y3, rh3 = _mid128(acc2[:_N], acc2[_N_PAD:_N_PAD + _N], W3_rel_p,
                      W3_root_p, b3_p.reshape(1, _H))
    acc3 = seg128(y3, rh3, src, dst)
    return _pool(acc3[:_N], acc3[_N_PAD:_N_PAD + _N],
                 batch.reshape(_GRID, 1, _BLK))


# speedup vs baseline: 1.0002x; 1.0002x over previous
import functools

import jax
import jax.numpy as jnp
from jax import lax
from jax.experimental import pallas as pl
from jax.experimental.pallas import tpu as pltpu
from jax.experimental.pallas import tpu_sc as plsc

_N = 10000
_E = 320000
_D = 128
_H = 128
_C = 16
_G = 64

_NC = 2
_NS = 16
_CHUNK = 128
_CPT = 80
_E_PAD = _NC * _NS * _CPT * _CHUNK
_N_PAD = 10240
_ROWS_PER_TILE = _N_PAD // _NS


def _seg_sum_body(feat, y_hbm, rh_hbm, src_hbm, dst_hbm, out_hbm,
                  acc_sh, src0, dst0, src1, dst1, rows0, rows1,
                  sem_i0, sem_i1, sem_g0, sem_g1):
    c = lax.axis_index("c")
    s = lax.axis_index("s")
    w = c * _NS + s
    srcs = (src0, src1)
    dsts = (dst0, dst1)
    rows = (rows0, rows1)
    isems = (sem_i0, sem_i1)
    gsems = (sem_g0, sem_g1)
    ebase = w * (_CPT * _CHUNK)

    base_r = s * _ROWS_PER_TILE
    last_full = (_N - (_NS - 1) * _ROWS_PER_TILE)

    @pl.when(s < _NS - 1)
    def _():
        pltpu.sync_copy(rh_hbm.at[pl.ds(base_r, _ROWS_PER_TILE)],
                        acc_sh.at[pl.ds(base_r, _ROWS_PER_TILE)])

    @pl.when(s == _NS - 1)
    def _():
        lr = (_NS - 1) * _ROWS_PER_TILE
        pltpu.sync_copy(rh_hbm.at[pl.ds(lr, last_full)],
                        acc_sh.at[pl.ds(lr, last_full)])

    plsc.subcore_barrier()

    def load_idx(j, b):
        off = pl.multiple_of(ebase + j * _CHUNK, 8)
        h0 = pltpu.async_copy(src_hbm.at[pl.ds(off, _CHUNK)], srcs[b], isems[b])
        h1 = pltpu.async_copy(dst_hbm.at[pl.ds(off, _CHUNK)], dsts[b], isems[b])
        return h0, h1

    def step(j, b, has_next):
        nb = 1 - b
        if has_next:
            i0, i1 = load_idx(j + 1, nb)
        pltpu.make_async_copy(y_hbm.at[srcs[b]], rows[b], gsems[b]).wait()
        if has_next:
            i0.wait()
            i1.wait()
            pltpu.async_copy(y_hbm.at[srcs[nb]], rows[nb], gsems[nb])
        pltpu.sync_copy(rows[b], acc_sh.at[dsts[b]], add=True)

    h0, h1 = load_idx(0, 0)
    h0.wait()
    h1.wait()
    pltpu.async_copy(y_hbm.at[src0], rows0, sem_g0)

    def body(k, carry):
        step(2 * k, 0, True)
        step(2 * k + 1, 1, True)
        return carry

    lax.fori_loop(0, (_CPT - 2) // 2, body, 0)
    step(_CPT - 2, 0, True)
    step(_CPT - 1, 1, False)
    plsc.subcore_barrier()

    row_off = c * _N_PAD + base_r
    pltpu.sync_copy(acc_sh.at[pl.ds(base_r, _ROWS_PER_TILE)],
                    out_hbm.at[pl.ds(row_off, _ROWS_PER_TILE)])


@functools.lru_cache(maxsize=None)
def _make_seg_sum(feat):
    mesh = plsc.VectorSubcoreMesh(core_axis_name="c", subcore_axis_name="s",
                                  num_cores=_NC, num_subcores=_NS)
    return pl.kernel(
        functools.partial(_seg_sum_body, feat),
        out_type=jax.ShapeDtypeStruct((_NC * _N_PAD, feat), jnp.float32),
        mesh=mesh,
        scratch_types=[
            pltpu.VMEM_SHARED((_N_PAD, feat), jnp.float32),
            pltpu.VMEM((_CHUNK,), jnp.int32),
            pltpu.VMEM((_CHUNK,), jnp.int32),
            pltpu.VMEM((_CHUNK,), jnp.int32),
            pltpu.VMEM((_CHUNK,), jnp.int32),
            pltpu.VMEM((_CHUNK, feat), jnp.float32),
            pltpu.VMEM((_CHUNK, feat), jnp.float32),
            pltpu.SemaphoreType.DMA,
            pltpu.SemaphoreType.DMA,
            pltpu.SemaphoreType.DMA,
            pltpu.SemaphoreType.DMA,
        ],
    )


_BLK = 1000
_GRID = _N // _BLK


def _front_body(x_ref, wrel_ref, wroot_ref, b_ref, y_ref, r_ref):
    xb = x_ref[...]
    y_ref[...] = jnp.dot(xb, wrel_ref[...], preferred_element_type=jnp.float32)
    r_ref[...] = 0.5 * (jnp.dot(xb, wroot_ref[...],
                                preferred_element_type=jnp.float32)
                        + b_ref[...])


def _mid_body(fo, acc_a_ref, acc_b_ref, wrel_ref, wroot_ref, b_ref,
              y_ref, r2_ref):
    h = jnp.maximum(acc_a_ref[...] + acc_b_ref[...], 0.0)
    y_ref[...] = jnp.dot(h, wrel_ref[...], preferred_element_type=jnp.float32)
    r2_ref[...] = 0.5 * (jnp.dot(h, wroot_ref[...],
                                 preferred_element_type=jnp.float32)
                         + b_ref[...])


def _make_front(fin, fo):
    return pl.pallas_call(
        _front_body,
        grid=(_GRID,),
        in_specs=[
            pl.BlockSpec((_BLK, fin), lambda i: (i, 0)),
            pl.BlockSpec((fin, fo), lambda i: (0, 0)),
            pl.BlockSpec((fin, fo), lambda i: (0, 0)),
            pl.BlockSpec((1, fo), lambda i: (0, 0)),
        ],
        out_specs=[
            pl.BlockSpec((_BLK, fo), lambda i: (i, 0)),
            pl.BlockSpec((_BLK, fo), lambda i: (i, 0)),
        ],
        out_shape=[
            jax.ShapeDtypeStruct((_N, fo), jnp.float32),
            jax.ShapeDtypeStruct((_N, fo), jnp.float32),
        ],
    )


def _make_mid(fin, fo):
    return pl.pallas_call(
        functools.partial(_mid_body, fo),
        grid=(_GRID,),
        in_specs=[
            pl.BlockSpec((_BLK, fin), lambda i: (i, 0)),
            pl.BlockSpec((_BLK, fin), lambda i: (i, 0)),
            pl.BlockSpec((fin, fo), lambda i: (0, 0)),
            pl.BlockSpec((fin, fo), lambda i: (0, 0)),
            pl.BlockSpec((1, fo), lambda i: (0, 0)),
        ],
        out_specs=[
            pl.BlockSpec((_BLK, fo), lambda i: (i, 0)),
            pl.BlockSpec((_BLK, fo), lambda i: (i, 0)),
        ],
        out_shape=[
            jax.ShapeDtypeStruct((_N, fo), jnp.float32),
            jax.ShapeDtypeStruct((_N, fo), jnp.float32),
        ],
    )


_front128 = _make_front(_D, _H)
_mid128 = _make_mid(_H, _H)


def _pool_body(acc_a_ref, acc_b_ref, batch_ref, out_ref, cnt_ref):
    i = pl.program_id(0)
    t = (acc_a_ref[...] + acc_b_ref[...])[:, :_C]
    bvec = batch_ref[0, 0, :]
    onehot = (bvec[:, None] ==
              lax.broadcasted_iota(jnp.int32, (_BLK, _G), 1)).astype(jnp.float32)
    dims = (((0,), (0,)), ((), ()))
    sums = lax.dot_general(onehot, t, dims, preferred_element_type=jnp.float32)
    cnts = lax.dot_general(onehot, jnp.ones((_BLK, _C), jnp.float32), dims,
                           preferred_element_type=jnp.float32)

    @pl.when(i == 0)
    def _():
        out_ref[...] = sums
        cnt_ref[...] = cnts

    @pl.when(i > 0)
    def _():
        out_ref[...] += sums
        cnt_ref[...] += cnts

    @pl.when(i == pl.num_programs(0) - 1)
    def _():
        out_ref[...] = out_ref[...] / jnp.maximum(cnt_ref[...], 1.0)


_pool = pl.pallas_call(
    _pool_body,
    grid=(_GRID,),
    in_specs=[
        pl.BlockSpec((_BLK, _H), lambda i: (i, 0)),
        pl.BlockSpec((_BLK, _H), lambda i: (i, 0)),
        pl.BlockSpec((1, 1, _BLK), lambda i: (i, 0, 0)),
    ],
    out_specs=pl.BlockSpec((_G, _C), lambda i: (0, 0)),
    out_shape=jax.ShapeDtypeStruct((_G, _C), jnp.float32),
    scratch_shapes=[pltpu.VMEM((_G, _C), jnp.float32)],
)


def kernel(x, edge_index, batch, W1_rel, W1_root, b1, W2_rel, W2_root, b2,
           W3_rel, W3_root, b3):
    npad = _E_PAD - _E
    pad_dst = _N + (jnp.arange(npad, dtype=jnp.int32) % (_N_PAD - _N))
    src = jnp.concatenate([edge_index[0],
                           jnp.zeros((npad,), jnp.int32)])
    dst = jnp.concatenate([edge_index[1], pad_dst])
    seg128 = _make_seg_sum(_H)
    W3_rel_p = jnp.pad(W3_rel, ((0, 0), (0, _H - _C)))
    W3_root_p = jnp.pad(W3_root, ((0, 0), (0, _H - _C)))
    b3_p = jnp.pad(b3, (0, _H - _C))
    y1, rh1 = _front128(x, W1_rel, W1_root, b1.reshape(1, _H))
    acc1 = seg128(y1, rh1, src, dst)
    y2, rh2 = _mid128(acc1[:_N], acc1[_N_PAD:_N_PAD + _N], W2_rel, W2_root,
                      b2.reshape(1, _H))
    acc2 = seg128(y2, rh2, src, dst)
    y3, rh3 = _mid128(acc2[:_N], acc2[_N_PAD:_N_PAD + _N], W3_rel_p,
                      W3_root_p, b3_p.reshape(1, _H))
    acc3 = seg128(y3, rh3, src, dst)
    return _pool(acc3[:_N], acc3[_N_PAD:_N_PAD + _N],
                 batch.reshape(_GRID, 1, _BLK))


# speedup vs baseline: 2.7059x; 2.7053x over previous
import functools

import jax
import jax.numpy as jnp
from jax import lax
from jax.experimental import pallas as pl
from jax.experimental.pallas import tpu as pltpu
from jax.experimental.pallas import tpu_sc as plsc

_N = 10000
_E = 320000
_D = 128
_H = 128
_C = 16
_G = 64

_NC = 2
_NS = 16
_CHUNK = 80
_CPT = _E // (_NC * _NS * _CHUNK)
_N_PAD = 10240
_ROWS_PER_TILE = _N_PAD // _NS


def _seg_sum_body(feat, y_hbm, rh_hbm, src_hbm, dst_hbm, out_hbm,
                  acc_sh, src0, dst0, src1, dst1, rows0, rows1,
                  sem_i0, sem_i1, sem_g0, sem_g1):
    c = lax.axis_index("c")
    s = lax.axis_index("s")
    w = c * _NS + s
    srcs = (src0, src1)
    dsts = (dst0, dst1)
    rows = (rows0, rows1)
    isems = (sem_i0, sem_i1)
    gsems = (sem_g0, sem_g1)
    ebase = w * (_CPT * _CHUNK)

    base_r = s * _ROWS_PER_TILE
    last_full = (_N - (_NS - 1) * _ROWS_PER_TILE)

    @pl.when(s < _NS - 1)
    def _():
        pltpu.sync_copy(rh_hbm.at[pl.ds(base_r, _ROWS_PER_TILE)],
                        acc_sh.at[pl.ds(base_r, _ROWS_PER_TILE)])

    @pl.when(s == _NS - 1)
    def _():
        lr = (_NS - 1) * _ROWS_PER_TILE
        pltpu.sync_copy(rh_hbm.at[pl.ds(lr, last_full)],
                        acc_sh.at[pl.ds(lr, last_full)])

    plsc.subcore_barrier()

    def load_idx(j, b):
        off = pl.multiple_of(ebase + j * _CHUNK, 8)
        h0 = pltpu.async_copy(src_hbm.at[pl.ds(off, _CHUNK)], srcs[b], isems[b])
        h1 = pltpu.async_copy(dst_hbm.at[pl.ds(off, _CHUNK)], dsts[b], isems[b])
        return h0, h1

    def step(j, b, has_next):
        nb = 1 - b
        if has_next:
            i0, i1 = load_idx(j + 1, nb)
        pltpu.make_async_copy(y_hbm.at[srcs[b]], rows[b], gsems[b]).wait()
        if has_next:
            i0.wait()
            i1.wait()
            pltpu.async_copy(y_hbm.at[srcs[nb]], rows[nb], gsems[nb])
        pltpu.sync_copy(rows[b], acc_sh.at[dsts[b]], add=True)

    h0, h1 = load_idx(0, 0)
    h0.wait()
    h1.wait()
    pltpu.async_copy(y_hbm.at[src0], rows0, sem_g0)

    def body(k, carry):
        step(2 * k, 0, True)
        step(2 * k + 1, 1, True)
        return carry

    lax.fori_loop(0, (_CPT - 2) // 2, body, 0)
    step(_CPT - 2, 0, True)
    step(_CPT - 1, 1, False)
    plsc.subcore_barrier()

    row_off = c * _N_PAD + base_r
    pltpu.sync_copy(acc_sh.at[pl.ds(base_r, _ROWS_PER_TILE)],
                    out_hbm.at[pl.ds(row_off, _ROWS_PER_TILE)])


@functools.lru_cache(maxsize=None)
def _make_seg_sum(feat):
    mesh = plsc.VectorSubcoreMesh(core_axis_name="c", subcore_axis_name="s",
                                  num_cores=_NC, num_subcores=_NS)
    return pl.kernel(
        functools.partial(_seg_sum_body, feat),
        out_type=jax.ShapeDtypeStruct((_NC * _N_PAD, feat), jnp.float32),
        mesh=mesh,
        scratch_types=[
            pltpu.VMEM_SHARED((_N_PAD, feat), jnp.float32),
            pltpu.VMEM((_CHUNK,), jnp.int32),
            pltpu.VMEM((_CHUNK,), jnp.int32),
            pltpu.VMEM((_CHUNK,), jnp.int32),
            pltpu.VMEM((_CHUNK,), jnp.int32),
            pltpu.VMEM((_CHUNK, feat), jnp.float32),
            pltpu.VMEM((_CHUNK, feat), jnp.float32),
            pltpu.SemaphoreType.DMA,
            pltpu.SemaphoreType.DMA,
            pltpu.SemaphoreType.DMA,
            pltpu.SemaphoreType.DMA,
        ],
    )


_BLK = 1000
_GRID = _N // _BLK


def _front_body(x_ref, wrel_ref, wroot_ref, b_ref, y_ref, r_ref):
    xb = x_ref[...]
    y_ref[...] = jnp.dot(xb, wrel_ref[...], preferred_element_type=jnp.float32)
    r_ref[...] = 0.5 * (jnp.dot(xb, wroot_ref[...],
                                preferred_element_type=jnp.float32)
                        + b_ref[...])


def _mid_body(fo, acc_a_ref, acc_b_ref, wrel_ref, wroot_ref, b_ref,
              y_ref, r2_ref):
    h = jnp.maximum(acc_a_ref[...] + acc_b_ref[...], 0.0)
    y_ref[...] = jnp.dot(h, wrel_ref[...], preferred_element_type=jnp.float32)
    r2_ref[...] = 0.5 * (jnp.dot(h, wroot_ref[...],
                                 preferred_element_type=jnp.float32)
                         + b_ref[...])


def _make_front(fin, fo):
    return pl.pallas_call(
        _front_body,
        grid=(_GRID,),
        in_specs=[
            pl.BlockSpec((_BLK, fin), lambda i: (i, 0)),
            pl.BlockSpec((fin, fo), lambda i: (0, 0)),
            pl.BlockSpec((fin, fo), lambda i: (0, 0)),
            pl.BlockSpec((1, fo), lambda i: (0, 0)),
        ],
        out_specs=[
            pl.BlockSpec((_BLK, fo), lambda i: (i, 0)),
            pl.BlockSpec((_BLK, fo), lambda i: (i, 0)),
        ],
        out_shape=[
            jax.ShapeDtypeStruct((_N_PAD, fo), jnp.float32),
            jax.ShapeDtypeStruct((_N_PAD, fo), jnp.float32),
        ],
    )


_BLKP = 1024
_GRIDP = _N_PAD // _BLKP


def _make_mid(fin, fo):
    return pl.pallas_call(
        functools.partial(_mid_body, fo),
        grid=(_GRIDP,),
        in_specs=[
            pl.BlockSpec((_BLKP, fin), lambda i: (i, 0)),
            pl.BlockSpec((_BLKP, fin), lambda i: (i + _GRIDP, 0)),
            pl.BlockSpec((fin, fo), lambda i: (0, 0)),
            pl.BlockSpec((fin, fo), lambda i: (0, 0)),
            pl.BlockSpec((1, fo), lambda i: (0, 0)),
        ],
        out_specs=[
            pl.BlockSpec((_BLKP, fo), lambda i: (i, 0)),
            pl.BlockSpec((_BLKP, fo), lambda i: (i, 0)),
        ],
        out_shape=[
            jax.ShapeDtypeStruct((_N_PAD, fo), jnp.float32),
            jax.ShapeDtypeStruct((_N_PAD, fo), jnp.float32),
        ],
    )


_front128 = _make_front(_D, _H)
_mid128 = _make_mid(_H, _H)


def _pool_body(acc_a_ref, acc_b_ref, batch_ref, out_ref, cnt_ref):
    i = pl.program_id(0)
    t = (acc_a_ref[...] + acc_b_ref[...])[:, :_C]
    bvec = batch_ref[0, 0, :]
    onehot = (bvec[:, None] ==
              lax.broadcasted_iota(jnp.int32, (_BLKP, _G), 1)).astype(jnp.float32)
    dims = (((0,), (0,)), ((), ()))
    sums = lax.dot_general(onehot, t, dims, preferred_element_type=jnp.float32)
    cnts = lax.dot_general(onehot, jnp.ones((_BLKP, _C), jnp.float32), dims,
                           preferred_element_type=jnp.float32)

    @pl.when(i == 0)
    def _():
        out_ref[...] = sums
        cnt_ref[...] = cnts

    @pl.when(i > 0)
    def _():
        out_ref[...] += sums
        cnt_ref[...] += cnts

    @pl.when(i == pl.num_programs(0) - 1)
    def _():
        out_ref[...] = out_ref[...] / jnp.maximum(cnt_ref[...], 1.0)


_pool = pl.pallas_call(
    _pool_body,
    grid=(_GRIDP,),
    in_specs=[
        pl.BlockSpec((_BLKP, _H), lambda i: (i, 0)),
        pl.BlockSpec((_BLKP, _H), lambda i: (i + _GRIDP, 0)),
        pl.BlockSpec((1, 1, _BLKP), lambda i: (i, 0, 0)),
    ],
    out_specs=pl.BlockSpec((_G, _C), lambda i: (0, 0)),
    out_shape=jax.ShapeDtypeStruct((_G, _C), jnp.float32),
    scratch_shapes=[pltpu.VMEM((_G, _C), jnp.float32)],
)


def kernel(x, edge_index, batch, W1_rel, W1_root, b1, W2_rel, W2_root, b2,
           W3_rel, W3_root, b3):
    src = edge_index[0]
    dst = edge_index[1]
    seg128 = _make_seg_sum(_H)
    W3_rel_p = jnp.pad(W3_rel, ((0, 0), (0, _H - _C)))
    W3_root_p = jnp.pad(W3_root, ((0, 0), (0, _H - _C)))
    b3_p = jnp.pad(b3, (0, _H - _C))
    batch_p = jnp.concatenate([batch, jnp.full((_N_PAD - _N,), _G, jnp.int32)])
    y1, rh1 = _front128(x, W1_rel, W1_root, b1.reshape(1, _H))
    acc1 = seg128(y1, rh1, src, dst)
    y2, rh2 = _mid128(acc1, acc1, W2_rel, W2_root, b2.reshape(1, _H))
    acc2 = seg128(y2, rh2, src, dst)
    y3, rh3 = _mid128(acc2, acc2, W3_rel_p, W3_root_p, b3_p.reshape(1, _H))
    acc3 = seg128(y3, rh3, src, dst)
    return _pool(acc3, acc3, batch_p.reshape(_GRIDP, 1, _BLKP))
